# baseline (speedup 1.0000x reference)
import jax
import jax.numpy as jnp
from jax.experimental import pallas as pl

_N = 21
_F = 128
_R_STATE, _R_ADJ, _R_W1, _R_W2, _R_END = 0, 21, 42, 170, 191


def _fused_fwd(pack_ref, Wf1_ref, Wf2_ref, Wf3_ref, Wpi_ref, Wf4_ref, Wv_ref,
               b1_ref, b2_ref, bf1_ref, bf2_ref, bf3_ref, bf4_ref,
               bpi_ref, bv_ref, out_ref):
    x0 = pack_ref[_R_STATE:_R_ADJ, :]
    a = (pack_ref[_R_ADJ:_R_W1, 0:_N]
         + jnp.eye(_N, dtype=jnp.float32))
    deg = jnp.sum(a, axis=1, keepdims=True)
    s = jnp.where(deg > 0, jax.lax.rsqrt(deg), 0.0)

    z = s * jnp.dot(x0, pack_ref[_R_W1:_R_W2, 0:_N],
                    preferred_element_type=jnp.float32)
    x = s * jnp.dot(a, z, preferred_element_type=jnp.float32) + b1_ref[...]
    z = s * jnp.dot(x, pack_ref[_R_W2:_R_END, 0:_N],
                    preferred_element_type=jnp.float32)
    x = s * jnp.dot(a, z, preferred_element_type=jnp.float32) + b2_ref[...]

    parts = [jnp.dot(x[i:i + 1, :], Wf1_ref[i * _N:(i + 1) * _N, :],
                     preferred_element_type=jnp.float32) for i in range(_N)]
    parts.append(bf1_ref[...])
    while len(parts) > 1:
        nxt = [parts[i] + parts[i + 1] for i in range(0, len(parts) - 1, 2)]
        if len(parts) % 2:
            nxt.append(parts[-1])
        parts = nxt
    h = jnp.maximum(parts[0], 0.0)

    h = jnp.maximum(
        jnp.dot(h, Wf2_ref[...], preferred_element_type=jnp.float32)
        + bf2_ref[...], 0.0)
    vx = jnp.maximum(
        jnp.dot(h, Wf3_ref[...], preferred_element_type=jnp.float32)
        + bf3_ref[...], 0.0)
    vx = jnp.maximum(
        jnp.dot(vx, Wf4_ref[...], preferred_element_type=jnp.float32)
        + bf4_ref[...], 0.0)

    out_ref[0:1, 0:64] = (
        jnp.dot(h, Wpi_ref[...], preferred_element_type=jnp.float32)
        + bpi_ref[...])
    out_ref[1:2, 0:1] = (
        jnp.dot(vx, Wv_ref[...], preferred_element_type=jnp.float32)
        + bv_ref[...])


def _pad128(m):
    return jnp.pad(m, ((0, 0), (0, 128 - m.shape[1])))


def kernel(state, adj, W1, b1, W2, b2, Wf1, bf1, Wf2, bf2, Wf3, bf3,
           Wf4, bf4, Wpi, bpi, Wv, bv):
    pack = jnp.concatenate([
        state[0],
        _pad128(adj[0]),
        _pad128(W1),
        _pad128(W2),
    ], axis=0)
    args = [pack, Wf1, Wf2, Wf3, Wpi, Wf4, Wv,
            b1.reshape(1, -1), b2.reshape(1, -1), bf1.reshape(1, -1),
            bf2.reshape(1, -1), bf3.reshape(1, -1), bf4.reshape(1, -1),
            bpi.reshape(1, -1), bv.reshape(1, 1)]
    full = lambda x: pl.BlockSpec(x.shape, lambda i: tuple(0 for _ in x.shape))
    out = pl.pallas_call(
        _fused_fwd,
        out_shape=jax.ShapeDtypeStruct((2, 128), jnp.float32),
        grid=(1,),
        in_specs=[full(x) for x in args],
        out_specs=pl.BlockSpec((2, 128), lambda i: (0, 0)),
    )(*args)
    return out[0, 0:64], out[1, 0:1]


# speedup vs baseline: 1.1535x; 1.1535x over previous
import jax
import jax.numpy as jnp
from jax.experimental import pallas as pl

_N = 21
_F = 128
_R_STATE, _R_ADJ, _R_W1, _R_W2, _R_END = 0, 21, 42, 170, 191


def _fused_fwd(pack_ref, Wf1_ref, Wf2_ref, Wf3_ref, Wpi_ref, Wf4_ref, Wv_ref,
               b1_ref, b2_ref, bf1_ref, bf2_ref, bf3_ref, bf4_ref,
               bpi_ref, bv_ref, pi_ref, v_ref):
    x0 = pack_ref[_R_STATE:_R_ADJ, :]
    a = (pack_ref[_R_ADJ:_R_W1, 0:_N]
         + jnp.eye(_N, dtype=jnp.float32))
    deg = jnp.sum(a, axis=1, keepdims=True)
    s = jnp.where(deg > 0, jax.lax.rsqrt(deg), 0.0)

    z = s * jnp.dot(x0, pack_ref[_R_W1:_R_W2, 0:_N],
                    preferred_element_type=jnp.float32)
    x = s * jnp.dot(a, z, preferred_element_type=jnp.float32) + b1_ref[...]
    z = s * jnp.dot(x, pack_ref[_R_W2:_R_END, 0:_N],
                    preferred_element_type=jnp.float32)
    x = s * jnp.dot(a, z, preferred_element_type=jnp.float32) + b2_ref[...]

    parts = [jnp.dot(x[i:i + 1, :], Wf1_ref[i * _N:(i + 1) * _N, :],
                     preferred_element_type=jnp.float32) for i in range(_N)]
    parts.append(bf1_ref[...])
    while len(parts) > 1:
        nxt = [parts[i] + parts[i + 1] for i in range(0, len(parts) - 1, 2)]
        if len(parts) % 2:
            nxt.append(parts[-1])
        parts = nxt
    h = jnp.maximum(parts[0], 0.0)

    h = jnp.maximum(
        jnp.dot(h, Wf2_ref[...], preferred_element_type=jnp.float32)
        + bf2_ref[...], 0.0)
    vx = jnp.maximum(
        jnp.dot(h, Wf3_ref[...], preferred_element_type=jnp.float32)
        + bf3_ref[...], 0.0)
    vx = jnp.maximum(
        jnp.dot(vx, Wf4_ref[...], preferred_element_type=jnp.float32)
        + bf4_ref[...], 0.0)

    pi_ref[...] = (jnp.dot(h, Wpi_ref[...], preferred_element_type=jnp.float32)
                   + bpi_ref[...])
    v_ref[...] = (jnp.dot(vx, Wv_ref[...], preferred_element_type=jnp.float32)
                  + bv_ref[...])


def _pad128(m):
    return jnp.pad(m, ((0, 0), (0, 128 - m.shape[1])))


def kernel(state, adj, W1, b1, W2, b2, Wf1, bf1, Wf2, bf2, Wf3, bf3,
           Wf4, bf4, Wpi, bpi, Wv, bv):
    pack = jnp.concatenate([
        state[0],
        _pad128(adj[0]),
        _pad128(W1),
        _pad128(W2),
    ], axis=0)
    args = [pack, Wf1, Wf2, Wf3, Wpi, Wf4, Wv,
            b1.reshape(1, -1), b2.reshape(1, -1), bf1.reshape(1, -1),
            bf2.reshape(1, -1), bf3.reshape(1, -1), bf4.reshape(1, -1),
            bpi.reshape(1, -1), bv.reshape(1, 1)]
    pi, v = pl.pallas_call(
        _fused_fwd,
        out_shape=(jax.ShapeDtypeStruct((1, 64), jnp.float32),
                   jax.ShapeDtypeStruct((1, 1), jnp.float32)),
    )(*args)
    return pi.reshape(64), v.reshape(1)
